# Initial kernel scaffold; baseline (speedup 1.0000x reference)
import jax
import jax.numpy as jnp
from jax.experimental import pallas as pl


def kernel(embeds, adj_edge_index, adj_edge_values):
    raise NotImplementedError("write your pallas kernel here")


# speedup vs baseline: 3.8834x; 3.8834x over previous
import functools

import jax
import jax.numpy as jnp
from jax import lax
from jax.experimental import pallas as pl
from jax.experimental.pallas import tpu as pltpu
from jax.experimental.pallas import tpu_sc as plsc

N = 10000
E = 320000
D = 128
MASK_DEPTH = 2
PATH_PROB = 0.5
NUM_MASK_CAND = 2048

NC, NS = 2, 16
NW = NC * NS
NPAD = 10240
EPAD = NW * 10240
EW = EPAD // NW
SUB = 128
SUPER = 2048
NSUPER = EW // SUPER
NSUB = SUPER // SUB
RPT = NPAD // NS
RPW = NPAD // NW
F32 = jnp.float32
I32 = jnp.int32

_MESH = plsc.VectorSubcoreMesh(
    core_axis_name="c", subcore_axis_name="s", num_cores=NC, num_subcores=NS)


def _zero16():
  return jnp.zeros((16,), F32)


def _spmm_body(with_t, x, rows_h, cols_h, vals_h, nprev_h,
               s_out, d_out, t_out,
               s_acc, d_acc, t_acc, rows_v, cols_v, vals_v, sidx,
               gbuf, nbuf, sem):
  c = lax.axis_index("c")
  s = lax.axis_index("s")
  w = c * NS + s
  ebase = w * EW

  def _zg(i, _):
    gbuf[i // 8, pl.ds((i % 8) * 16, 16)] = _zero16()
    return 0
  lax.fori_loop(0, SUB * 8, _zg, 0)

  def _zn(i, _):
    nbuf[pl.ds(i * 16, 16)] = _zero16()
    return 0
  lax.fori_loop(0, SUB // 16, _zn, 0)

  rz = s * RPT
  for k in range(RPT // SUB):
    pltpu.sync_copy(gbuf, s_acc.at[pl.ds(rz + k * SUB, SUB)])
    pltpu.sync_copy(nbuf, d_acc.at[pl.ds(rz + k * SUB, SUB)])
    if with_t:
      pltpu.sync_copy(nbuf, t_acc.at[pl.ds(rz + k * SUB, SUB)])

  trash = jnp.full((16,), I32(0)) + (N + s).astype(I32)

  plsc.subcore_barrier()

  def _super(sc, _):
    eb = ebase + sc * SUPER
    pltpu.sync_copy(rows_h.at[pl.ds(eb, SUPER)], rows_v)
    pltpu.sync_copy(cols_h.at[pl.ds(eb, SUPER)], cols_v)
    pltpu.sync_copy(vals_h.at[pl.ds(eb, SUPER)], vals_v)

    def _mk(i, _):
      r = rows_v[pl.ds(i * 16, 16)]
      v = vals_v[pl.ds(i * 16, 16)]
      si = jnp.where(v != 0.0, r, trash)
      sidx[i // 8, pl.ds((i % 8) * 16, 16)] = si
      return 0
    lax.fori_loop(0, SUPER // 16, _mk, 0)

    def _step(j, _):
      idx = cols_v.at[pl.ds(j * SUB, SUB)]
      pltpu.async_copy(x.at[idx], gbuf, sem).wait()
      pltpu.sync_copy(gbuf, s_acc.at[sidx.at[j]], add=True)
      pltpu.sync_copy(vals_v.at[pl.ds(j * SUB, SUB)], d_acc.at[sidx.at[j]],
                      add=True)
      if with_t:
        pltpu.async_copy(nprev_h.at[idx], nbuf, sem).wait()
        pltpu.sync_copy(nbuf, t_acc.at[sidx.at[j]], add=True)
      return 0
    lax.fori_loop(0, NSUB, _step, 0)
    return 0
  lax.fori_loop(0, NSUPER, _super, 0)

  plsc.subcore_barrier()

  pltpu.sync_copy(s_acc.at[pl.ds(rz, RPT)], s_out.at[c].at[pl.ds(rz, RPT)])
  pltpu.sync_copy(d_acc.at[pl.ds(rz, RPT)], d_out.at[c].at[pl.ds(rz, RPT)])
  if with_t:
    pltpu.sync_copy(t_acc.at[pl.ds(rz, RPT)], t_out.at[c].at[pl.ds(rz, RPT)])


def _make_spmm(with_t):
  body = functools.partial(_spmm_body, with_t)
  return pl.kernel(
      body,
      out_type=(
          jax.ShapeDtypeStruct((NC, NPAD, D), F32),
          jax.ShapeDtypeStruct((NC, NPAD), F32),
          jax.ShapeDtypeStruct((NC, NPAD), F32),
      ),
      mesh=_MESH,
      scratch_types=[
          pltpu.VMEM_SHARED((NPAD, D), F32),
          pltpu.VMEM_SHARED((NPAD,), F32),
          pltpu.VMEM_SHARED((NPAD,), F32),
          pltpu.VMEM((SUPER,), I32),
          pltpu.VMEM((SUPER,), I32),
          pltpu.VMEM((SUPER,), F32),
          pltpu.VMEM((NSUB, SUB), I32),
          pltpu.VMEM((SUB, D), F32),
          pltpu.VMEM((SUB,), F32),
          pltpu.SemaphoreType.DMA,
      ],
  )


def _fin0_body(sp, dp, x_h, e_out, d_out,
               b0, b1, bx, db0, db1, sem):
  del sem
  c = lax.axis_index("c")
  s = lax.axis_index("s")
  w = c * NS + s
  rbase = w * RPW

  @pl.when(c == 0)
  def _scalars():
    sb = s * RPT
    pltpu.sync_copy(dp.at[0].at[pl.ds(sb, RPT)], db0)
    pltpu.sync_copy(dp.at[1].at[pl.ds(sb, RPT)], db1)

    def _dsum(i, _):
      db0[pl.ds(i * 16, 16)] = (db0[pl.ds(i * 16, 16)]
                                + db1[pl.ds(i * 16, 16)])
      return 0
    lax.fori_loop(0, RPT // 16, _dsum, 0)
    pltpu.sync_copy(db0, d_out.at[pl.ds(sb, RPT)])

  for half in range(2):
    rb = rbase + half * 160
    pltpu.sync_copy(sp.at[0].at[pl.ds(rb, 160)], b0)
    pltpu.sync_copy(sp.at[1].at[pl.ds(rb, 160)], b1)
    pltpu.sync_copy(x_h.at[pl.ds(rb, 160)], bx)

    def _ew(i, _):
      r = i // 8
      j = (i % 8) * 16
      b0[r, pl.ds(j, 16)] = (b0[r, pl.ds(j, 16)] + b1[r, pl.ds(j, 16)]
                             - bx[r, pl.ds(j, 16)])
      return 0
    lax.fori_loop(0, 160 * 8, _ew, 0)
    pltpu.sync_copy(b0, e_out.at[pl.ds(rb, 160)])


_fin0 = pl.kernel(
    _fin0_body,
    out_type=(
        jax.ShapeDtypeStruct((NPAD, D), F32),
        jax.ShapeDtypeStruct((NPAD,), F32),
    ),
    mesh=_MESH,
    scratch_types=[
        pltpu.VMEM((160, D), F32),
        pltpu.VMEM((160, D), F32),
        pltpu.VMEM((160, D), F32),
        pltpu.VMEM((RPT,), F32),
        pltpu.VMEM((RPT,), F32),
        pltpu.SemaphoreType.DMA,
    ],
)


def _fink_body(sp, dp, tp, eprev_h, dprev_h, nprev_h,
               e_out, d_out, n_out,
               b0, b1, bx, db0, db1, tb0, tb1, dpv, sem):
  del sem
  c = lax.axis_index("c")
  s = lax.axis_index("s")
  w = c * NS + s
  rbase = w * RPW

  @pl.when(c == 0)
  def _scalars():
    sb = s * RPT
    pltpu.sync_copy(dp.at[0].at[pl.ds(sb, RPT)], db0)
    pltpu.sync_copy(dp.at[1].at[pl.ds(sb, RPT)], db1)
    pltpu.sync_copy(tp.at[0].at[pl.ds(sb, RPT)], tb0)
    pltpu.sync_copy(tp.at[1].at[pl.ds(sb, RPT)], tb1)
    pltpu.sync_copy(dprev_h.at[pl.ds(sb, RPT)], dpv)
    pltpu.sync_copy(nprev_h.at[pl.ds(sb, RPT)], db1)

    def _sc(i, _):
      o = i * 16
      nk = (tb0[pl.ds(o, 16)] + tb1[pl.ds(o, 16)]
            - db1[pl.ds(o, 16)] - dpv[pl.ds(o, 16)])
      tb0[pl.ds(o, 16)] = nk
      return 0
    lax.fori_loop(0, RPT // 16, _sc, 0)
    pltpu.sync_copy(tb0, n_out.at[pl.ds(sb, RPT)])

    pltpu.sync_copy(dp.at[1].at[pl.ds(sb, RPT)], db1)

    def _ds(i, _):
      o = i * 16
      db0[pl.ds(o, 16)] = db0[pl.ds(o, 16)] + db1[pl.ds(o, 16)]
      return 0
    lax.fori_loop(0, RPT // 16, _ds, 0)
    pltpu.sync_copy(db0, d_out.at[pl.ds(sb, RPT)])

  pltpu.sync_copy(dprev_h.at[pl.ds((w // 2) * RPT, RPT)], dpv)
  dof = (w % 2) * RPW

  for half in range(2):
    rb = rbase + half * 160
    pltpu.sync_copy(sp.at[0].at[pl.ds(rb, 160)], b0)
    pltpu.sync_copy(sp.at[1].at[pl.ds(rb, 160)], b1)
    pltpu.sync_copy(eprev_h.at[pl.ds(rb, 160)], bx)

    def _ew(g, _):
      dvec = dpv[pl.ds(dof + half * 160 + g * 16, 16)]
      for l in range(16):
        dprev = dvec[l] + F32(1.0)
        r = g * 16 + l
        for j in range(8):
          b0[r, pl.ds(j * 16, 16)] = (b0[r, pl.ds(j * 16, 16)]
                                      + b1[r, pl.ds(j * 16, 16)]
                                      - dprev * bx[r, pl.ds(j * 16, 16)])
      return 0
    lax.fori_loop(0, 10, _ew, 0)
    pltpu.sync_copy(b0, e_out.at[pl.ds(rb, 160)])


_fink = pl.kernel(
    _fink_body,
    out_type=(
        jax.ShapeDtypeStruct((NPAD, D), F32),
        jax.ShapeDtypeStruct((NPAD,), F32),
        jax.ShapeDtypeStruct((NPAD,), F32),
    ),
    mesh=_MESH,
    scratch_types=[
        pltpu.VMEM((160, D), F32),
        pltpu.VMEM((160, D), F32),
        pltpu.VMEM((160, D), F32),
        pltpu.VMEM((RPT,), F32),
        pltpu.VMEM((RPT,), F32),
        pltpu.VMEM((RPT,), F32),
        pltpu.VMEM((RPT,), F32),
        pltpu.VMEM((RPT,), F32),
        pltpu.SemaphoreType.DMA,
    ],
)


def _score_body(e0_ref, e1_ref, e2_ref, n_ref, x_ref, g_ref, out_ref):
  esum = (e0_ref[...] + e1_ref[...]) + e2_ref[...]
  nsum = n_ref[...]
  sub = esum / (nsum + F32(1e-8))
  snrm = jnp.sqrt(jnp.sum(sub * sub, axis=-1, keepdims=True))
  sub = sub / jnp.maximum(snrm, F32(1e-12))
  x = x_ref[...]
  xnrm = jnp.sqrt(jnp.sum(x * x, axis=-1, keepdims=True))
  xn = x / jnp.maximum(xnrm, F32(1e-12))
  out_ref[...] = jnp.sum(sub * xn, axis=-1, keepdims=True) + g_ref[...]


def _score_call(e0, e1, e2, nsum2d, x, g2d):
  return pl.pallas_call(
      _score_body,
      out_shape=jax.ShapeDtypeStruct((NPAD, 1), F32),
  )(e0, e1, e2, nsum2d, x, g2d)


_spmm_not = _make_spmm(False)
_spmm_t = _make_spmm(True)


def kernel(embeds, adj_edge_index, adj_edge_values):
  embeds = embeds[:N]
  rows = adj_edge_index[0]
  cols = adj_edge_index[1]
  vals0 = adj_edge_values

  key = jax.random.key(42)
  vals = vals0
  level_vals = [vals0]
  for i in range(MASK_DEPTH):
    key, kd = jax.random.split(key)
    keep = PATH_PROB ** (i + 1)
    msk = jnp.floor(jax.random.uniform(kd, (E,)) + keep)
    vals = vals * msk
    level_vals.append(vals)
  key, kn = jax.random.split(key)
  noise = jax.random.uniform(kn, (N,))
  gumbel = -jnp.log(-jnp.log(noise + 1e-20) + 1e-20)

  epad = EPAD - E
  rows_p = jnp.concatenate([rows, jnp.zeros((epad,), I32)])
  cols_p = jnp.concatenate([cols, jnp.zeros((epad,), I32)])
  lv = [jnp.concatenate([v.astype(F32), jnp.zeros((epad,), F32)])
        for v in level_vals]

  x0 = jnp.zeros((NPAD, D), F32).at[:N].set(embeds)
  gum_pad = jnp.zeros((NPAD, 1), F32).at[:N, 0].set(gumbel)
  zeros_n = jnp.zeros((NPAD,), F32)

  s0p, d0p, _ = _spmm_not(x0, rows_p, cols_p, lv[0], zeros_n)
  e0, d0 = _fin0(s0p, d0p, x0)
  s1p, d1p, t1p = _spmm_t(e0, rows_p, cols_p, lv[1], d0)
  e1, d1, n1 = _fink(s1p, d1p, t1p, e0, d0, d0)
  s2p, d2p, t2p = _spmm_t(e1, rows_p, cols_p, lv[2], n1)
  e2, _, n2 = _fink(s2p, d2p, t2p, e1, d1, n1)

  nsum = (d0 + n1) + n2
  scores2d = _score_call(e0, e1, e2, nsum.reshape(NPAD, 1), x0, gum_pad)
  scores = scores2d[:N, 0]
  _, candidates = lax.top_k(scores, NUM_MASK_CAND)
  return scores, candidates


# speedup vs baseline: 4.6170x; 1.1889x over previous
import functools

import jax
import jax.numpy as jnp
from jax import lax
from jax.experimental import pallas as pl
from jax.experimental.pallas import tpu as pltpu
from jax.experimental.pallas import tpu_sc as plsc

N = 10000
E = 320000
D = 128
MASK_DEPTH = 2
PATH_PROB = 0.5
NUM_MASK_CAND = 2048

NC, NS = 2, 16
NW = NC * NS
NPAD = 10240
EPAD = NW * 10240
EW = EPAD // NW
SUB = 128
SUPER = 2048
NSUPER = EW // SUPER
NSUB = SUPER // SUB
RPT = NPAD // NS
RPW = NPAD // NW
F32 = jnp.float32
I32 = jnp.int32

_MESH = plsc.VectorSubcoreMesh(
    core_axis_name="c", subcore_axis_name="s", num_cores=NC, num_subcores=NS)


def _zero16():
  return jnp.zeros((16,), F32)


def _spmm_body(with_t, x, rows_h, cols_h, vals_h, nprev_h,
               s_out, d_out, t_out,
               s_acc, d_acc, t_acc, rows_v, cols_v, vals_v, sidx,
               gbuf0, gbuf1, nbuf0, nbuf1,
               gsem0, gsem1, ssem0, ssem1, dsem0, dsem1,
               nsem0, nsem1, tsem0, tsem1):
  c = lax.axis_index("c")
  s = lax.axis_index("s")
  w = c * NS + s
  ebase = w * EW
  gbufs = (gbuf0, gbuf1)
  nbufs = (nbuf0, nbuf1)
  gsem = (gsem0, gsem1)
  ssem = (ssem0, ssem1)
  dsem = (dsem0, dsem1)
  nsem = (nsem0, nsem1)
  tsem = (tsem0, tsem1)

  def _zg(i, _):
    gbuf0[i // 8, pl.ds((i % 8) * 16, 16)] = _zero16()
    return 0
  lax.fori_loop(0, SUB * 8, _zg, 0)

  def _zn(i, _):
    nbuf0[pl.ds(i * 16, 16)] = _zero16()
    return 0
  lax.fori_loop(0, SUB // 16, _zn, 0)

  rz = s * RPT
  for k in range(RPT // SUB):
    pltpu.sync_copy(gbuf0, s_acc.at[pl.ds(rz + k * SUB, SUB)])
    pltpu.sync_copy(nbuf0, d_acc.at[pl.ds(rz + k * SUB, SUB)])
    if with_t:
      pltpu.sync_copy(nbuf0, t_acc.at[pl.ds(rz + k * SUB, SUB)])

  trash = jnp.full((16,), I32(0)) + (N + s).astype(I32)

  plsc.subcore_barrier()

  def _super(sc, _):
    eb = ebase + sc * SUPER
    pltpu.sync_copy(rows_h.at[pl.ds(eb, SUPER)], rows_v)
    pltpu.sync_copy(cols_h.at[pl.ds(eb, SUPER)], cols_v)
    pltpu.sync_copy(vals_h.at[pl.ds(eb, SUPER)], vals_v)

    def _mk(i, _):
      r = rows_v[pl.ds(i * 16, 16)]
      v = vals_v[pl.ds(i * 16, 16)]
      si = jnp.where(v != 0.0, r, trash)
      sidx[i // 8, pl.ds((i % 8) * 16, 16)] = si
      return 0
    lax.fori_loop(0, SUPER // 16, _mk, 0)

    gd = [None, None]
    nd = [None, None]
    sd = [None, None]

    def _fire(j):
      p = j & 1
      idx = cols_v.at[pl.ds(j * SUB, SUB)]
      gd[p] = pltpu.async_copy(x.at[idx], gbufs[p], gsem[p])
      if with_t:
        nd[p] = pltpu.async_copy(nprev_h.at[idx], nbufs[p], nsem[p])

    _fire(0)
    for j in range(NSUB):
      p = j & 1
      q = 1 - p
      if j + 1 < NSUB:
        if sd[q] is not None:
          for dsc in sd[q]:
            dsc.wait()
          sd[q] = None
        _fire(j + 1)
      gd[p].wait()
      lst = [
          pltpu.async_copy(gbufs[p], s_acc.at[sidx.at[j]], ssem[p], add=True),
          pltpu.async_copy(vals_v.at[pl.ds(j * SUB, SUB)],
                           d_acc.at[sidx.at[j]], dsem[p], add=True),
      ]
      if with_t:
        nd[p].wait()
        lst.append(pltpu.async_copy(nbufs[p], t_acc.at[sidx.at[j]],
                                    tsem[p], add=True))
      sd[p] = lst
    for p in (0, 1):
      if sd[p] is not None:
        for dsc in sd[p]:
          dsc.wait()
    return 0
  lax.fori_loop(0, NSUPER, _super, 0)

  plsc.subcore_barrier()

  pltpu.sync_copy(s_acc.at[pl.ds(rz, RPT)], s_out.at[c].at[pl.ds(rz, RPT)])
  pltpu.sync_copy(d_acc.at[pl.ds(rz, RPT)], d_out.at[c].at[pl.ds(rz, RPT)])
  if with_t:
    pltpu.sync_copy(t_acc.at[pl.ds(rz, RPT)], t_out.at[c].at[pl.ds(rz, RPT)])


def _make_spmm(with_t):
  body = functools.partial(_spmm_body, with_t)
  return pl.kernel(
      body,
      out_type=(
          jax.ShapeDtypeStruct((NC, NPAD, D), F32),
          jax.ShapeDtypeStruct((NC, NPAD), F32),
          jax.ShapeDtypeStruct((NC, NPAD), F32),
      ),
      mesh=_MESH,
      scratch_types=[
          pltpu.VMEM_SHARED((NPAD, D), F32),
          pltpu.VMEM_SHARED((NPAD,), F32),
          pltpu.VMEM_SHARED((NPAD,), F32),
          pltpu.VMEM((SUPER,), I32),
          pltpu.VMEM((SUPER,), I32),
          pltpu.VMEM((SUPER,), F32),
          pltpu.VMEM((NSUB, SUB), I32),
          pltpu.VMEM((SUB, D), F32),
          pltpu.VMEM((SUB, D), F32),
          pltpu.VMEM((SUB,), F32),
          pltpu.VMEM((SUB,), F32),
      ] + [pltpu.SemaphoreType.DMA] * 10,
  )


def _fin0_body(sp, dp, x_h, e_out, d_out,
               b0, b1, bx, db0, db1, sem):
  del sem
  c = lax.axis_index("c")
  s = lax.axis_index("s")
  w = c * NS + s
  rbase = w * RPW

  @pl.when(c == 0)
  def _scalars():
    sb = s * RPT
    pltpu.sync_copy(dp.at[0].at[pl.ds(sb, RPT)], db0)
    pltpu.sync_copy(dp.at[1].at[pl.ds(sb, RPT)], db1)

    def _dsum(i, _):
      db0[pl.ds(i * 16, 16)] = (db0[pl.ds(i * 16, 16)]
                                + db1[pl.ds(i * 16, 16)])
      return 0
    lax.fori_loop(0, RPT // 16, _dsum, 0)
    pltpu.sync_copy(db0, d_out.at[pl.ds(sb, RPT)])

  for half in range(2):
    rb = rbase + half * 160
    pltpu.sync_copy(sp.at[0].at[pl.ds(rb, 160)], b0)
    pltpu.sync_copy(sp.at[1].at[pl.ds(rb, 160)], b1)
    pltpu.sync_copy(x_h.at[pl.ds(rb, 160)], bx)

    def _ew(i, _):
      r = i // 8
      j = (i % 8) * 16
      b0[r, pl.ds(j, 16)] = (b0[r, pl.ds(j, 16)] + b1[r, pl.ds(j, 16)]
                             - bx[r, pl.ds(j, 16)])
      return 0
    lax.fori_loop(0, 160 * 8, _ew, 0)
    pltpu.sync_copy(b0, e_out.at[pl.ds(rb, 160)])


_fin0 = pl.kernel(
    _fin0_body,
    out_type=(
        jax.ShapeDtypeStruct((NPAD, D), F32),
        jax.ShapeDtypeStruct((NPAD,), F32),
    ),
    mesh=_MESH,
    scratch_types=[
        pltpu.VMEM((160, D), F32),
        pltpu.VMEM((160, D), F32),
        pltpu.VMEM((160, D), F32),
        pltpu.VMEM((RPT,), F32),
        pltpu.VMEM((RPT,), F32),
        pltpu.SemaphoreType.DMA,
    ],
)


def _fink_body(sp, dp, tp, eprev_h, dprev_h, nprev_h,
               e_out, d_out, n_out,
               b0, b1, bx, db0, db1, tb0, tb1, dpv, sem):
  del sem
  c = lax.axis_index("c")
  s = lax.axis_index("s")
  w = c * NS + s
  rbase = w * RPW

  @pl.when(c == 0)
  def _scalars():
    sb = s * RPT
    pltpu.sync_copy(dp.at[0].at[pl.ds(sb, RPT)], db0)
    pltpu.sync_copy(dp.at[1].at[pl.ds(sb, RPT)], db1)
    pltpu.sync_copy(tp.at[0].at[pl.ds(sb, RPT)], tb0)
    pltpu.sync_copy(tp.at[1].at[pl.ds(sb, RPT)], tb1)
    pltpu.sync_copy(dprev_h.at[pl.ds(sb, RPT)], dpv)
    pltpu.sync_copy(nprev_h.at[pl.ds(sb, RPT)], db1)

    def _sc(i, _):
      o = i * 16
      nk = (tb0[pl.ds(o, 16)] + tb1[pl.ds(o, 16)]
            - db1[pl.ds(o, 16)] - dpv[pl.ds(o, 16)])
      tb0[pl.ds(o, 16)] = nk
      return 0
    lax.fori_loop(0, RPT // 16, _sc, 0)
    pltpu.sync_copy(tb0, n_out.at[pl.ds(sb, RPT)])

    pltpu.sync_copy(dp.at[1].at[pl.ds(sb, RPT)], db1)

    def _ds(i, _):
      o = i * 16
      db0[pl.ds(o, 16)] = db0[pl.ds(o, 16)] + db1[pl.ds(o, 16)]
      return 0
    lax.fori_loop(0, RPT // 16, _ds, 0)
    pltpu.sync_copy(db0, d_out.at[pl.ds(sb, RPT)])

  pltpu.sync_copy(dprev_h.at[pl.ds((w // 2) * RPT, RPT)], dpv)
  dof = (w % 2) * RPW

  for half in range(2):
    rb = rbase + half * 160
    pltpu.sync_copy(sp.at[0].at[pl.ds(rb, 160)], b0)
    pltpu.sync_copy(sp.at[1].at[pl.ds(rb, 160)], b1)
    pltpu.sync_copy(eprev_h.at[pl.ds(rb, 160)], bx)

    def _ew(g, _):
      dvec = dpv[pl.ds(dof + half * 160 + g * 16, 16)]
      for l in range(16):
        dprev = dvec[l]
        r = g * 16 + l
        for j in range(8):
          xv = bx[r, pl.ds(j * 16, 16)]
          b0[r, pl.ds(j * 16, 16)] = (b0[r, pl.ds(j * 16, 16)]
                                      + b1[r, pl.ds(j * 16, 16)]
                                      - xv - dprev * xv)
      return 0
    lax.fori_loop(0, 10, _ew, 0)
    pltpu.sync_copy(b0, e_out.at[pl.ds(rb, 160)])


_fink = pl.kernel(
    _fink_body,
    out_type=(
        jax.ShapeDtypeStruct((NPAD, D), F32),
        jax.ShapeDtypeStruct((NPAD,), F32),
        jax.ShapeDtypeStruct((NPAD,), F32),
    ),
    mesh=_MESH,
    scratch_types=[
        pltpu.VMEM((160, D), F32),
        pltpu.VMEM((160, D), F32),
        pltpu.VMEM((160, D), F32),
        pltpu.VMEM((RPT,), F32),
        pltpu.VMEM((RPT,), F32),
        pltpu.VMEM((RPT,), F32),
        pltpu.VMEM((RPT,), F32),
        pltpu.VMEM((RPT,), F32),
        pltpu.SemaphoreType.DMA,
    ],
)


def _score_body(s2p_ref, t2p_ref, e0_ref, e1_ref, d0_ref, d1_ref, n1_ref,
                x_ref, g_ref, out_ref):
  e0 = e0_ref[...]
  e1 = e1_ref[...]
  d1 = d1_ref[...]
  e2 = (s2p_ref[0] + s2p_ref[1]) - e1 - d1 * e1
  n2 = (t2p_ref[0] + t2p_ref[1]) - n1_ref[...] - d1
  esum = (e0 + e1) + e2
  nsum = (d0_ref[...] + n1_ref[...]) + n2
  sub = esum / (nsum + F32(1e-8))
  snrm = jnp.sqrt(jnp.sum(sub * sub, axis=-1, keepdims=True))
  sub = sub / jnp.maximum(snrm, F32(1e-12))
  x = x_ref[...]
  xnrm = jnp.sqrt(jnp.sum(x * x, axis=-1, keepdims=True))
  xn = x / jnp.maximum(xnrm, F32(1e-12))
  out_ref[...] = jnp.sum(sub * xn, axis=-1, keepdims=True) + g_ref[...]


_SBLK = 2048


def _score_call(s2p, t2p, e0, e1, d0, d1, n1, x, g2d):
  mat = pl.BlockSpec((NC, _SBLK, D), lambda i: (0, i, 0))
  rowm = pl.BlockSpec((_SBLK, D), lambda i: (i, 0))
  col3 = pl.BlockSpec((NC, _SBLK, 1), lambda i: (0, i, 0))
  col = pl.BlockSpec((_SBLK, 1), lambda i: (i, 0))
  return pl.pallas_call(
      _score_body,
      grid=(NPAD // _SBLK,),
      in_specs=[mat, col3, rowm, rowm, col, col, col, rowm, col],
      out_specs=col,
      out_shape=jax.ShapeDtypeStruct((NPAD, 1), F32),
  )(s2p, t2p.reshape(NC, NPAD, 1), e0, e1, d0.reshape(NPAD, 1),
    d1.reshape(NPAD, 1), n1.reshape(NPAD, 1), x, g2d)


_spmm_not = _make_spmm(False)
_spmm_t = _make_spmm(True)


def kernel(embeds, adj_edge_index, adj_edge_values):
  embeds = embeds[:N]
  rows = adj_edge_index[0]
  cols = adj_edge_index[1]
  vals0 = adj_edge_values

  key = jax.random.key(42)
  vals = vals0
  level_vals = [vals0]
  for i in range(MASK_DEPTH):
    key, kd = jax.random.split(key)
    keep = PATH_PROB ** (i + 1)
    msk = jnp.floor(jax.random.uniform(kd, (E,)) + keep)
    vals = vals * msk
    level_vals.append(vals)
  key, kn = jax.random.split(key)
  noise = jax.random.uniform(kn, (N,))
  gumbel = -jnp.log(-jnp.log(noise + 1e-20) + 1e-20)

  epad = EPAD - E
  rows_p = jnp.concatenate([rows, jnp.zeros((epad,), I32)])
  cols_p = jnp.concatenate([cols, jnp.zeros((epad,), I32)])
  lv = [jnp.concatenate([v.astype(F32), jnp.zeros((epad,), F32)])
        for v in level_vals]

  x0 = jnp.zeros((NPAD, D), F32).at[:N].set(embeds)
  gum_pad = jnp.zeros((NPAD, 1), F32).at[:N, 0].set(gumbel)
  zeros_n = jnp.zeros((NPAD,), F32)

  s0p, d0p, _ = _spmm_not(x0, rows_p, cols_p, lv[0], zeros_n)
  e0, d0 = _fin0(s0p, d0p, x0)
  s1p, d1p, t1p = _spmm_t(e0, rows_p, cols_p, lv[1], d0)
  e1, d1, n1 = _fink(s1p, d1p, t1p, e0, d0, d0)
  s2p, _, t2p = _spmm_t(e1, rows_p, cols_p, lv[2], n1)

  scores2d = _score_call(s2p, t2p, e0, e1, d0, d1, n1, x0, gum_pad)
  scores = scores2d[:N, 0]
  _, candidates = lax.top_k(scores, NUM_MASK_CAND)
  return scores, candidates


# speedup vs baseline: 4.6364x; 1.0042x over previous
import functools

import jax
import jax.numpy as jnp
from jax import lax
from jax.experimental import pallas as pl
from jax.experimental.pallas import tpu as pltpu
from jax.experimental.pallas import tpu_sc as plsc

N = 10000
E = 320000
D = 128
MASK_DEPTH = 2
PATH_PROB = 0.5
NUM_MASK_CAND = 2048

NC, NS = 2, 16
NW = NC * NS
NPAD = 10240
EPAD = NW * 10240
EW = EPAD // NW
SUB = 64
SUPER = 2048
NSUPER = EW // SUPER
NSUB = SUPER // SUB
NBUF = 4
RPT = NPAD // NS
RPW = NPAD // NW
F32 = jnp.float32
I32 = jnp.int32

_MESH = plsc.VectorSubcoreMesh(
    core_axis_name="c", subcore_axis_name="s", num_cores=NC, num_subcores=NS)


def _zero16():
  return jnp.zeros((16,), F32)


def _spmm_body(with_t, x, rows_h, cols_h, vals_h, nprev_h,
               s_out, d_out, t_out,
               s_acc, d_acc, t_acc, rows_v, cols_v, vals_v, sidx,
               gbuf0, gbuf1, gbuf2, gbuf3, nbuf0, nbuf1, nbuf2, nbuf3,
               gsem0, gsem1, gsem2, gsem3, ssem0, ssem1, ssem2, ssem3,
               dsem0, dsem1, dsem2, dsem3, nsem0, nsem1, nsem2, nsem3,
               tsem0, tsem1, tsem2, tsem3):
  c = lax.axis_index("c")
  s = lax.axis_index("s")
  w = c * NS + s
  ebase = w * EW
  gbufs = (gbuf0, gbuf1, gbuf2, gbuf3)
  nbufs = (nbuf0, nbuf1, nbuf2, nbuf3)
  gsem = (gsem0, gsem1, gsem2, gsem3)
  ssem = (ssem0, ssem1, ssem2, ssem3)
  dsem = (dsem0, dsem1, dsem2, dsem3)
  nsem = (nsem0, nsem1, nsem2, nsem3)
  tsem = (tsem0, tsem1, tsem2, tsem3)

  def _zg(i, _):
    gbuf0[i // 8, pl.ds((i % 8) * 16, 16)] = _zero16()
    return 0
  lax.fori_loop(0, SUB * 8, _zg, 0)

  def _zn(i, _):
    nbuf0[pl.ds(i * 16, 16)] = _zero16()
    return 0
  lax.fori_loop(0, SUB // 16, _zn, 0)

  rz = s * RPT
  for k in range(RPT // SUB):
    pltpu.sync_copy(gbuf0, s_acc.at[pl.ds(rz + k * SUB, SUB)])
    pltpu.sync_copy(nbuf0, d_acc.at[pl.ds(rz + k * SUB, SUB)])
    if with_t:
      pltpu.sync_copy(nbuf0, t_acc.at[pl.ds(rz + k * SUB, SUB)])

  trash = jnp.full((16,), I32(0)) + (N + s).astype(I32)

  plsc.subcore_barrier()

  def _super(sc, _):
    eb = ebase + sc * SUPER
    pltpu.sync_copy(rows_h.at[pl.ds(eb, SUPER)], rows_v)
    pltpu.sync_copy(cols_h.at[pl.ds(eb, SUPER)], cols_v)
    pltpu.sync_copy(vals_h.at[pl.ds(eb, SUPER)], vals_v)

    def _mk(i, _):
      r = rows_v[pl.ds(i * 16, 16)]
      v = vals_v[pl.ds(i * 16, 16)]
      si = jnp.where(v != 0.0, r, trash)
      sidx[i // 4, pl.ds((i % 4) * 16, 16)] = si
      return 0
    lax.fori_loop(0, SUPER // 16, _mk, 0)

    gd = [None] * NBUF
    nd = [None] * NBUF
    sd = [None] * NBUF

    def _fire(j):
      p = j % NBUF
      idx = cols_v.at[pl.ds(j * SUB, SUB)]
      gd[p] = pltpu.async_copy(x.at[idx], gbufs[p], gsem[p])
      if with_t:
        nd[p] = pltpu.async_copy(nprev_h.at[idx], nbufs[p], nsem[p])

    def _finish(j):
      p = j % NBUF
      gd[p].wait()
      sd[p] = [
          pltpu.async_copy(gbufs[p], s_acc.at[sidx.at[j]], ssem[p], add=True),
          pltpu.async_copy(vals_v.at[pl.ds(j * SUB, SUB)],
                           d_acc.at[sidx.at[j]], dsem[p], add=True),
      ]
      if with_t:
        nd[p].wait()
        sd[p].append(pltpu.async_copy(nbufs[p], t_acc.at[sidx.at[j]],
                                      tsem[p], add=True))

    for j in range(NSUB):
      p = j % NBUF
      if sd[p] is not None:
        for dsc in sd[p]:
          dsc.wait()
        sd[p] = None
      _fire(j)
      if j >= 1:
        _finish(j - 1)
    _finish(NSUB - 1)
    for p in range(NBUF):
      if sd[p] is not None:
        for dsc in sd[p]:
          dsc.wait()
    return 0

  lax.fori_loop(0, NSUPER, _super, 0)

  plsc.subcore_barrier()

  pltpu.sync_copy(s_acc.at[pl.ds(rz, RPT)], s_out.at[c].at[pl.ds(rz, RPT)])
  pltpu.sync_copy(d_acc.at[pl.ds(rz, RPT)], d_out.at[c].at[pl.ds(rz, RPT)])
  if with_t:
    pltpu.sync_copy(t_acc.at[pl.ds(rz, RPT)], t_out.at[c].at[pl.ds(rz, RPT)])


def _make_spmm(with_t):
  body = functools.partial(_spmm_body, with_t)
  return pl.kernel(
      body,
      out_type=(
          jax.ShapeDtypeStruct((NC, NPAD, D), F32),
          jax.ShapeDtypeStruct((NC, NPAD), F32),
          jax.ShapeDtypeStruct((NC, NPAD), F32),
      ),
      mesh=_MESH,
      scratch_types=[
          pltpu.VMEM_SHARED((NPAD, D), F32),
          pltpu.VMEM_SHARED((NPAD,), F32),
          pltpu.VMEM_SHARED((NPAD,), F32),
          pltpu.VMEM((SUPER,), I32),
          pltpu.VMEM((SUPER,), I32),
          pltpu.VMEM((SUPER,), F32),
          pltpu.VMEM((NSUB, SUB), I32),
      ] + [pltpu.VMEM((SUB, D), F32)] * NBUF
      + [pltpu.VMEM((SUB,), F32)] * NBUF
      + [pltpu.SemaphoreType.DMA] * (5 * NBUF),
  )


def _fin0_body(sp, dp, x_h, e_out, d_out,
               b0, b1, bx, db0, db1, sem):
  del sem
  c = lax.axis_index("c")
  s = lax.axis_index("s")
  w = c * NS + s
  rbase = w * RPW

  @pl.when(c == 0)
  def _scalars():
    sb = s * RPT
    pltpu.sync_copy(dp.at[0].at[pl.ds(sb, RPT)], db0)
    pltpu.sync_copy(dp.at[1].at[pl.ds(sb, RPT)], db1)

    def _dsum(i, _):
      db0[pl.ds(i * 16, 16)] = (db0[pl.ds(i * 16, 16)]
                                + db1[pl.ds(i * 16, 16)])
      return 0
    lax.fori_loop(0, RPT // 16, _dsum, 0)
    pltpu.sync_copy(db0, d_out.at[pl.ds(sb, RPT)])

  for half in range(2):
    rb = rbase + half * 160
    pltpu.sync_copy(sp.at[0].at[pl.ds(rb, 160)], b0)
    pltpu.sync_copy(sp.at[1].at[pl.ds(rb, 160)], b1)
    pltpu.sync_copy(x_h.at[pl.ds(rb, 160)], bx)

    def _ew(i, _):
      r = i // 8
      j = (i % 8) * 16
      b0[r, pl.ds(j, 16)] = (b0[r, pl.ds(j, 16)] + b1[r, pl.ds(j, 16)]
                             - bx[r, pl.ds(j, 16)])
      return 0
    lax.fori_loop(0, 160 * 8, _ew, 0)
    pltpu.sync_copy(b0, e_out.at[pl.ds(rb, 160)])


_fin0 = pl.kernel(
    _fin0_body,
    out_type=(
        jax.ShapeDtypeStruct((NPAD, D), F32),
        jax.ShapeDtypeStruct((NPAD,), F32),
    ),
    mesh=_MESH,
    scratch_types=[
        pltpu.VMEM((160, D), F32),
        pltpu.VMEM((160, D), F32),
        pltpu.VMEM((160, D), F32),
        pltpu.VMEM((RPT,), F32),
        pltpu.VMEM((RPT,), F32),
        pltpu.SemaphoreType.DMA,
    ],
)


def _fink_body(sp, dp, tp, eprev_h, dprev_h, nprev_h,
               e_out, d_out, n_out,
               b0, b1, bx, db0, db1, tb0, tb1, dpv, sem):
  del sem
  c = lax.axis_index("c")
  s = lax.axis_index("s")
  w = c * NS + s
  rbase = w * RPW

  @pl.when(c == 0)
  def _scalars():
    sb = s * RPT
    pltpu.sync_copy(dp.at[0].at[pl.ds(sb, RPT)], db0)
    pltpu.sync_copy(dp.at[1].at[pl.ds(sb, RPT)], db1)
    pltpu.sync_copy(tp.at[0].at[pl.ds(sb, RPT)], tb0)
    pltpu.sync_copy(tp.at[1].at[pl.ds(sb, RPT)], tb1)
    pltpu.sync_copy(dprev_h.at[pl.ds(sb, RPT)], dpv)
    pltpu.sync_copy(nprev_h.at[pl.ds(sb, RPT)], db1)

    def _sc(i, _):
      o = i * 16
      nk = (tb0[pl.ds(o, 16)] + tb1[pl.ds(o, 16)]
            - db1[pl.ds(o, 16)] - dpv[pl.ds(o, 16)])
      tb0[pl.ds(o, 16)] = nk
      return 0
    lax.fori_loop(0, RPT // 16, _sc, 0)
    pltpu.sync_copy(tb0, n_out.at[pl.ds(sb, RPT)])

    pltpu.sync_copy(dp.at[1].at[pl.ds(sb, RPT)], db1)

    def _ds(i, _):
      o = i * 16
      db0[pl.ds(o, 16)] = db0[pl.ds(o, 16)] + db1[pl.ds(o, 16)]
      return 0
    lax.fori_loop(0, RPT // 16, _ds, 0)
    pltpu.sync_copy(db0, d_out.at[pl.ds(sb, RPT)])

  pltpu.sync_copy(dprev_h.at[pl.ds((w // 2) * RPT, RPT)], dpv)
  dof = (w % 2) * RPW

  for half in range(2):
    rb = rbase + half * 160
    pltpu.sync_copy(sp.at[0].at[pl.ds(rb, 160)], b0)
    pltpu.sync_copy(sp.at[1].at[pl.ds(rb, 160)], b1)
    pltpu.sync_copy(eprev_h.at[pl.ds(rb, 160)], bx)

    def _ew(g, _):
      dvec = dpv[pl.ds(dof + half * 160 + g * 16, 16)]
      for l in range(16):
        dprev = dvec[l]
        r = g * 16 + l
        for j in range(8):
          xv = bx[r, pl.ds(j * 16, 16)]
          b0[r, pl.ds(j * 16, 16)] = (b0[r, pl.ds(j * 16, 16)]
                                      + b1[r, pl.ds(j * 16, 16)]
                                      - xv - dprev * xv)
      return 0
    lax.fori_loop(0, 10, _ew, 0)
    pltpu.sync_copy(b0, e_out.at[pl.ds(rb, 160)])


_fink = pl.kernel(
    _fink_body,
    out_type=(
        jax.ShapeDtypeStruct((NPAD, D), F32),
        jax.ShapeDtypeStruct((NPAD,), F32),
        jax.ShapeDtypeStruct((NPAD,), F32),
    ),
    mesh=_MESH,
    scratch_types=[
        pltpu.VMEM((160, D), F32),
        pltpu.VMEM((160, D), F32),
        pltpu.VMEM((160, D), F32),
        pltpu.VMEM((RPT,), F32),
        pltpu.VMEM((RPT,), F32),
        pltpu.VMEM((RPT,), F32),
        pltpu.VMEM((RPT,), F32),
        pltpu.VMEM((RPT,), F32),
        pltpu.SemaphoreType.DMA,
    ],
)


def _score_body(s2p_ref, t2p_ref, e0_ref, e1_ref, d0_ref, d1_ref, n1_ref,
                x_ref, g_ref, out_ref):
  e0 = e0_ref[...]
  e1 = e1_ref[...]
  d1 = d1_ref[...]
  e2 = (s2p_ref[0] + s2p_ref[1]) - e1 - d1 * e1
  n2 = (t2p_ref[0] + t2p_ref[1]) - n1_ref[...] - d1
  esum = (e0 + e1) + e2
  nsum = (d0_ref[...] + n1_ref[...]) + n2
  sub = esum / (nsum + F32(1e-8))
  snrm = jnp.sqrt(jnp.sum(sub * sub, axis=-1, keepdims=True))
  sub = sub / jnp.maximum(snrm, F32(1e-12))
  x = x_ref[...]
  xnrm = jnp.sqrt(jnp.sum(x * x, axis=-1, keepdims=True))
  xn = x / jnp.maximum(xnrm, F32(1e-12))
  out_ref[...] = jnp.sum(sub * xn, axis=-1, keepdims=True) + g_ref[...]


_SBLK = 2048


def _score_call(s2p, t2p, e0, e1, d0, d1, n1, x, g2d):
  mat = pl.BlockSpec((NC, _SBLK, D), lambda i: (0, i, 0))
  rowm = pl.BlockSpec((_SBLK, D), lambda i: (i, 0))
  col3 = pl.BlockSpec((NC, _SBLK, 1), lambda i: (0, i, 0))
  col = pl.BlockSpec((_SBLK, 1), lambda i: (i, 0))
  return pl.pallas_call(
      _score_body,
      grid=(NPAD // _SBLK,),
      in_specs=[mat, col3, rowm, rowm, col, col, col, rowm, col],
      out_specs=col,
      out_shape=jax.ShapeDtypeStruct((NPAD, 1), F32),
  )(s2p, t2p.reshape(NC, NPAD, 1), e0, e1, d0.reshape(NPAD, 1),
    d1.reshape(NPAD, 1), n1.reshape(NPAD, 1), x, g2d)


_spmm_not = _make_spmm(False)
_spmm_t = _make_spmm(True)


def kernel(embeds, adj_edge_index, adj_edge_values):
  embeds = embeds[:N]
  rows = adj_edge_index[0]
  cols = adj_edge_index[1]
  vals0 = adj_edge_values

  key = jax.random.key(42)
  vals = vals0
  level_vals = [vals0]
  for i in range(MASK_DEPTH):
    key, kd = jax.random.split(key)
    keep = PATH_PROB ** (i + 1)
    msk = jnp.floor(jax.random.uniform(kd, (E,)) + keep)
    vals = vals * msk
    level_vals.append(vals)
  key, kn = jax.random.split(key)
  noise = jax.random.uniform(kn, (N,))
  gumbel = -jnp.log(-jnp.log(noise + 1e-20) + 1e-20)

  epad = EPAD - E
  rows_p = jnp.concatenate([rows, jnp.zeros((epad,), I32)])
  cols_p = jnp.concatenate([cols, jnp.zeros((epad,), I32)])
  lv = [jnp.concatenate([v.astype(F32), jnp.zeros((epad,), F32)])
        for v in level_vals]

  x0 = jnp.zeros((NPAD, D), F32).at[:N].set(embeds)
  gum_pad = jnp.zeros((NPAD, 1), F32).at[:N, 0].set(gumbel)
  zeros_n = jnp.zeros((NPAD,), F32)

  s0p, d0p, _ = _spmm_not(x0, rows_p, cols_p, lv[0], zeros_n)
  e0, d0 = _fin0(s0p, d0p, x0)
  s1p, d1p, t1p = _spmm_t(e0, rows_p, cols_p, lv[1], d0)
  e1, d1, n1 = _fink(s1p, d1p, t1p, e0, d0, d0)
  s2p, _, t2p = _spmm_t(e1, rows_p, cols_p, lv[2], n1)

  scores2d = _score_call(s2p, t2p, e0, e1, d0, d1, n1, x0, gum_pad)
  scores = scores2d[:N, 0]
  _, candidates = lax.top_k(scores, NUM_MASK_CAND)
  return scores, candidates


# speedup vs baseline: 4.6403x; 1.0008x over previous
import functools

import jax
import jax.numpy as jnp
from jax import lax
from jax.experimental import pallas as pl
from jax.experimental.pallas import tpu as pltpu
from jax.experimental.pallas import tpu_sc as plsc

N = 10000
E = 320000
D = 128
MASK_DEPTH = 2
PATH_PROB = 0.5
NUM_MASK_CAND = 2048

NC, NS = 2, 16
NW = NC * NS
NPAD = 10240
EPAD = NW * 10240
EW = EPAD // NW
SUB = 64
SUPER = 2048
NSUPER = EW // SUPER
NSUB = SUPER // SUB
NBUF = 4
RPT = NPAD // NS
RPW = NPAD // NW
F32 = jnp.float32
I32 = jnp.int32

_MESH = plsc.VectorSubcoreMesh(
    core_axis_name="c", subcore_axis_name="s", num_cores=NC, num_subcores=NS)


def _zero16():
  return jnp.zeros((16,), F32)


def _spmm_body(with_t, with_d, x, rows_h, cols_h, vals_h, nprev_h,
               s_out, d_out, t_out,
               s_acc, d_acc, t_acc, rows_v, cols_v, vals_v, sidx,
               gbuf0, gbuf1, gbuf2, gbuf3, nbuf0, nbuf1, nbuf2, nbuf3,
               gsem0, gsem1, gsem2, gsem3, ssem0, ssem1, ssem2, ssem3,
               dsem0, dsem1, dsem2, dsem3, nsem0, nsem1, nsem2, nsem3,
               tsem0, tsem1, tsem2, tsem3):
  c = lax.axis_index("c")
  s = lax.axis_index("s")
  w = c * NS + s
  ebase = w * EW
  gbufs = (gbuf0, gbuf1, gbuf2, gbuf3)
  nbufs = (nbuf0, nbuf1, nbuf2, nbuf3)
  gsem = (gsem0, gsem1, gsem2, gsem3)
  ssem = (ssem0, ssem1, ssem2, ssem3)
  dsem = (dsem0, dsem1, dsem2, dsem3)
  nsem = (nsem0, nsem1, nsem2, nsem3)
  tsem = (tsem0, tsem1, tsem2, tsem3)

  def _zg(i, _):
    gbuf0[i // 8, pl.ds((i % 8) * 16, 16)] = _zero16()
    return 0
  lax.fori_loop(0, SUB * 8, _zg, 0)

  def _zn(i, _):
    nbuf0[pl.ds(i * 16, 16)] = _zero16()
    return 0
  lax.fori_loop(0, SUB // 16, _zn, 0)

  rz = s * RPT
  for k in range(RPT // SUB):
    pltpu.sync_copy(gbuf0, s_acc.at[pl.ds(rz + k * SUB, SUB)])
    if with_d:
      pltpu.sync_copy(nbuf0, d_acc.at[pl.ds(rz + k * SUB, SUB)])
    if with_t:
      pltpu.sync_copy(nbuf0, t_acc.at[pl.ds(rz + k * SUB, SUB)])

  trash = jnp.full((16,), I32(0)) + (N + s).astype(I32)

  plsc.subcore_barrier()

  def _super(sc, _):
    eb = ebase + sc * SUPER
    pltpu.sync_copy(rows_h.at[pl.ds(eb, SUPER)], rows_v)
    pltpu.sync_copy(cols_h.at[pl.ds(eb, SUPER)], cols_v)
    pltpu.sync_copy(vals_h.at[pl.ds(eb, SUPER)], vals_v)

    def _mk(i, _):
      r = rows_v[pl.ds(i * 16, 16)]
      v = vals_v[pl.ds(i * 16, 16)]
      si = jnp.where(v != 0.0, r, trash)
      sidx[i // 4, pl.ds((i % 4) * 16, 16)] = si
      return 0
    lax.fori_loop(0, SUPER // 16, _mk, 0)

    gd = [None] * NBUF
    nd = [None] * NBUF
    sd = [None] * NBUF

    def _fire(j):
      p = j % NBUF
      idx = cols_v.at[pl.ds(j * SUB, SUB)]
      gd[p] = pltpu.async_copy(x.at[idx], gbufs[p], gsem[p])
      if with_t:
        nd[p] = pltpu.async_copy(nprev_h.at[idx], nbufs[p], nsem[p])

    def _finish(j):
      p = j % NBUF
      gd[p].wait()
      sd[p] = [
          pltpu.async_copy(gbufs[p], s_acc.at[sidx.at[j]], ssem[p], add=True),
      ]
      if with_d:
        sd[p].append(pltpu.async_copy(vals_v.at[pl.ds(j * SUB, SUB)],
                                      d_acc.at[sidx.at[j]], dsem[p],
                                      add=True))
      if with_t:
        nd[p].wait()
        sd[p].append(pltpu.async_copy(nbufs[p], t_acc.at[sidx.at[j]],
                                      tsem[p], add=True))

    for j in range(NSUB):
      p = j % NBUF
      if sd[p] is not None:
        for dsc in sd[p]:
          dsc.wait()
        sd[p] = None
      _fire(j)
      if j >= 1:
        _finish(j - 1)
    _finish(NSUB - 1)
    for p in range(NBUF):
      if sd[p] is not None:
        for dsc in sd[p]:
          dsc.wait()
    return 0

  lax.fori_loop(0, NSUPER, _super, 0)

  plsc.subcore_barrier()

  pltpu.sync_copy(s_acc.at[pl.ds(rz, RPT)], s_out.at[c].at[pl.ds(rz, RPT)])
  if with_d:
    pltpu.sync_copy(d_acc.at[pl.ds(rz, RPT)], d_out.at[c].at[pl.ds(rz, RPT)])
  if with_t:
    pltpu.sync_copy(t_acc.at[pl.ds(rz, RPT)], t_out.at[c].at[pl.ds(rz, RPT)])


def _make_spmm(with_t, with_d=True):
  body = functools.partial(_spmm_body, with_t, with_d)
  return pl.kernel(
      body,
      out_type=(
          jax.ShapeDtypeStruct((NC, NPAD, D), F32),
          jax.ShapeDtypeStruct((NC, NPAD), F32),
          jax.ShapeDtypeStruct((NC, NPAD), F32),
      ),
      mesh=_MESH,
      scratch_types=[
          pltpu.VMEM_SHARED((NPAD, D), F32),
          pltpu.VMEM_SHARED((NPAD,), F32),
          pltpu.VMEM_SHARED((NPAD,), F32),
          pltpu.VMEM((SUPER,), I32),
          pltpu.VMEM((SUPER,), I32),
          pltpu.VMEM((SUPER,), F32),
          pltpu.VMEM((NSUB, SUB), I32),
      ] + [pltpu.VMEM((SUB, D), F32)] * NBUF
      + [pltpu.VMEM((SUB,), F32)] * NBUF
      + [pltpu.SemaphoreType.DMA] * (5 * NBUF),
  )


def _fin0_body(sp, dp, x_h, e_out, d_out,
               b0, b1, bx, db0, db1, sem):
  del sem
  c = lax.axis_index("c")
  s = lax.axis_index("s")
  w = c * NS + s
  rbase = w * RPW

  @pl.when(c == 0)
  def _scalars():
    sb = s * RPT
    pltpu.sync_copy(dp.at[0].at[pl.ds(sb, RPT)], db0)
    pltpu.sync_copy(dp.at[1].at[pl.ds(sb, RPT)], db1)

    def _dsum(i, _):
      db0[pl.ds(i * 16, 16)] = (db0[pl.ds(i * 16, 16)]
                                + db1[pl.ds(i * 16, 16)])
      return 0
    lax.fori_loop(0, RPT // 16, _dsum, 0)
    pltpu.sync_copy(db0, d_out.at[pl.ds(sb, RPT)])

  for half in range(2):
    rb = rbase + half * 160
    pltpu.sync_copy(sp.at[0].at[pl.ds(rb, 160)], b0)
    pltpu.sync_copy(sp.at[1].at[pl.ds(rb, 160)], b1)
    pltpu.sync_copy(x_h.at[pl.ds(rb, 160)], bx)

    def _ew(i, _):
      r = i // 8
      j = (i % 8) * 16
      b0[r, pl.ds(j, 16)] = (b0[r, pl.ds(j, 16)] + b1[r, pl.ds(j, 16)]
                             - bx[r, pl.ds(j, 16)])
      return 0
    lax.fori_loop(0, 160 * 8, _ew, 0)
    pltpu.sync_copy(b0, e_out.at[pl.ds(rb, 160)])


_fin0 = pl.kernel(
    _fin0_body,
    out_type=(
        jax.ShapeDtypeStruct((NPAD, D), F32),
        jax.ShapeDtypeStruct((NPAD,), F32),
    ),
    mesh=_MESH,
    scratch_types=[
        pltpu.VMEM((160, D), F32),
        pltpu.VMEM((160, D), F32),
        pltpu.VMEM((160, D), F32),
        pltpu.VMEM((RPT,), F32),
        pltpu.VMEM((RPT,), F32),
        pltpu.SemaphoreType.DMA,
    ],
)


def _fink_body(sp, dp, tp, eprev_h, dprev_h, nprev_h,
               e_out, d_out, n_out,
               b0, b1, bx, db0, db1, tb0, tb1, dpv, sem):
  del sem
  c = lax.axis_index("c")
  s = lax.axis_index("s")
  w = c * NS + s
  rbase = w * RPW

  @pl.when(c == 0)
  def _scalars():
    sb = s * RPT
    pltpu.sync_copy(dp.at[0].at[pl.ds(sb, RPT)], db0)
    pltpu.sync_copy(dp.at[1].at[pl.ds(sb, RPT)], db1)
    pltpu.sync_copy(tp.at[0].at[pl.ds(sb, RPT)], tb0)
    pltpu.sync_copy(tp.at[1].at[pl.ds(sb, RPT)], tb1)
    pltpu.sync_copy(dprev_h.at[pl.ds(sb, RPT)], dpv)
    pltpu.sync_copy(nprev_h.at[pl.ds(sb, RPT)], db1)

    def _sc(i, _):
      o = i * 16
      nk = (tb0[pl.ds(o, 16)] + tb1[pl.ds(o, 16)]
            - db1[pl.ds(o, 16)] - dpv[pl.ds(o, 16)])
      tb0[pl.ds(o, 16)] = nk
      return 0
    lax.fori_loop(0, RPT // 16, _sc, 0)
    pltpu.sync_copy(tb0, n_out.at[pl.ds(sb, RPT)])

    pltpu.sync_copy(dp.at[1].at[pl.ds(sb, RPT)], db1)

    def _ds(i, _):
      o = i * 16
      db0[pl.ds(o, 16)] = db0[pl.ds(o, 16)] + db1[pl.ds(o, 16)]
      return 0
    lax.fori_loop(0, RPT // 16, _ds, 0)
    pltpu.sync_copy(db0, d_out.at[pl.ds(sb, RPT)])

  pltpu.sync_copy(dprev_h.at[pl.ds((w // 2) * RPT, RPT)], dpv)
  dof = (w % 2) * RPW

  for half in range(2):
    rb = rbase + half * 160
    pltpu.sync_copy(sp.at[0].at[pl.ds(rb, 160)], b0)
    pltpu.sync_copy(sp.at[1].at[pl.ds(rb, 160)], b1)
    pltpu.sync_copy(eprev_h.at[pl.ds(rb, 160)], bx)

    def _ew(g, _):
      dvec = dpv[pl.ds(dof + half * 160 + g * 16, 16)]
      for l in range(16):
        dprev = dvec[l]
        r = g * 16 + l
        for j in range(8):
          xv = bx[r, pl.ds(j * 16, 16)]
          b0[r, pl.ds(j * 16, 16)] = (b0[r, pl.ds(j * 16, 16)]
                                      + b1[r, pl.ds(j * 16, 16)]
                                      - xv - dprev * xv)
      return 0
    lax.fori_loop(0, 10, _ew, 0)
    pltpu.sync_copy(b0, e_out.at[pl.ds(rb, 160)])


_fink = pl.kernel(
    _fink_body,
    out_type=(
        jax.ShapeDtypeStruct((NPAD, D), F32),
        jax.ShapeDtypeStruct((NPAD,), F32),
        jax.ShapeDtypeStruct((NPAD,), F32),
    ),
    mesh=_MESH,
    scratch_types=[
        pltpu.VMEM((160, D), F32),
        pltpu.VMEM((160, D), F32),
        pltpu.VMEM((160, D), F32),
        pltpu.VMEM((RPT,), F32),
        pltpu.VMEM((RPT,), F32),
        pltpu.VMEM((RPT,), F32),
        pltpu.VMEM((RPT,), F32),
        pltpu.VMEM((RPT,), F32),
        pltpu.SemaphoreType.DMA,
    ],
)


def _score_body(s2p_ref, t2p_ref, e0_ref, e1_ref, d0_ref, d1_ref, n1_ref,
                x_ref, g_ref, out_ref):
  e0 = e0_ref[...]
  e1 = e1_ref[...]
  d1 = d1_ref[...]
  e2 = (s2p_ref[0] + s2p_ref[1]) - e1 - d1 * e1
  n2 = (t2p_ref[0] + t2p_ref[1]) - n1_ref[...] - d1
  esum = (e0 + e1) + e2
  nsum = (d0_ref[...] + n1_ref[...]) + n2
  sub = esum / (nsum + F32(1e-8))
  snrm = jnp.sqrt(jnp.sum(sub * sub, axis=-1, keepdims=True))
  sub = sub / jnp.maximum(snrm, F32(1e-12))
  x = x_ref[...]
  xnrm = jnp.sqrt(jnp.sum(x * x, axis=-1, keepdims=True))
  xn = x / jnp.maximum(xnrm, F32(1e-12))
  out_ref[...] = jnp.sum(sub * xn, axis=-1, keepdims=True) + g_ref[...]


_SBLK = 2048


def _score_call(s2p, t2p, e0, e1, d0, d1, n1, x, g2d):
  mat = pl.BlockSpec((NC, _SBLK, D), lambda i: (0, i, 0))
  rowm = pl.BlockSpec((_SBLK, D), lambda i: (i, 0))
  col3 = pl.BlockSpec((NC, _SBLK, 1), lambda i: (0, i, 0))
  col = pl.BlockSpec((_SBLK, 1), lambda i: (i, 0))
  return pl.pallas_call(
      _score_body,
      grid=(NPAD // _SBLK,),
      in_specs=[mat, col3, rowm, rowm, col, col, col, rowm, col],
      out_specs=col,
      out_shape=jax.ShapeDtypeStruct((NPAD, 1), F32),
  )(s2p, t2p.reshape(NC, NPAD, 1), e0, e1, d0.reshape(NPAD, 1),
    d1.reshape(NPAD, 1), n1.reshape(NPAD, 1), x, g2d)


_spmm_not = _make_spmm(False)
_spmm_t = _make_spmm(True)
_spmm_t_nod = _make_spmm(True, with_d=False)


def kernel(embeds, adj_edge_index, adj_edge_values):
  embeds = embeds[:N]
  rows = adj_edge_index[0]
  cols = adj_edge_index[1]
  vals0 = adj_edge_values

  key = jax.random.key(42)
  vals = vals0
  level_vals = [vals0]
  for i in range(MASK_DEPTH):
    key, kd = jax.random.split(key)
    keep = PATH_PROB ** (i + 1)
    msk = jnp.floor(jax.random.uniform(kd, (E,)) + keep)
    vals = vals * msk
    level_vals.append(vals)
  key, kn = jax.random.split(key)
  noise = jax.random.uniform(kn, (N,))
  gumbel = -jnp.log(-jnp.log(noise + 1e-20) + 1e-20)

  epad = EPAD - E
  rows_p = jnp.concatenate([rows, jnp.zeros((epad,), I32)])
  cols_p = jnp.concatenate([cols, jnp.zeros((epad,), I32)])
  lv = [jnp.concatenate([v.astype(F32), jnp.zeros((epad,), F32)])
        for v in level_vals]

  x0 = jnp.zeros((NPAD, D), F32).at[:N].set(embeds)
  gum_pad = jnp.zeros((NPAD, 1), F32).at[:N, 0].set(gumbel)
  zeros_n = jnp.zeros((NPAD,), F32)

  s0p, d0p, _ = _spmm_not(x0, rows_p, cols_p, lv[0], zeros_n)
  e0, d0 = _fin0(s0p, d0p, x0)
  s1p, d1p, t1p = _spmm_t(e0, rows_p, cols_p, lv[1], d0)
  e1, d1, n1 = _fink(s1p, d1p, t1p, e0, d0, d0)
  s2p, _, t2p = _spmm_t_nod(e1, rows_p, cols_p, lv[2], n1)

  scores2d = _score_call(s2p, t2p, e0, e1, d0, d1, n1, x0, gum_pad)
  scores = scores2d[:N, 0]
  _, candidates = lax.top_k(scores, NUM_MASK_CAND)
  return scores, candidates
